# baseline (speedup 1.0000x reference)
import functools

import jax
import jax.numpy as jnp
from jax import lax
from jax.experimental import pallas as pl
from jax.experimental.pallas import tpu as pltpu
from jax.experimental.pallas import tpu_sc as plsc

N = 10000
E = 160000
B = 4
K = 4
R = 4 * N
N2 = 2 * N

CH = 128
EP = 163840
NCHUNKS = EP // CH
NSUB = 16
CPT = NCHUNKS // NSUB
_ROWSPLIT = tuple((t * 632, 632 if t < 15 else N - 15 * 632) for t in range(NSUB))

BLK = 800
EPS = 1e-5


def _chunks(tn):
  out = []
  o = 0
  while o < tn:
    c = min(128, tn - o)
    out.append((o, c))
    o += c
  return out


@functools.cache
def _make_cheb3(F):
  mesh = plsc.VectorSubcoreMesh(core_axis_name="c", subcore_axis_name="s")

  def body(x_hbm, src_hbm, dst_hbm, w_hbm, z_hbm, t1_hbm, t2_hbm, a3_hbm,
           src_v, dst_v, w_v, rows0, rows1, acc, sem0, sem1):
    cid = lax.axis_index("c")
    sid = lax.axis_index("s")
    c0 = sid * CPT
    pltpu.sync_copy(src_hbm.at[pl.ds(c0, CPT)], src_v)
    pltpu.sync_copy(dst_hbm.at[pl.ds(c0, CPT)], dst_v)
    pltpu.sync_copy(w_hbm.at[pl.ds(c0, CPT)], w_v)
    off = jnp.broadcast_to(cid * N, (16,))

    def shift_body(i, c2):
      for j in range(CH // 16):
        sl = pl.ds(j * 16, 16)
        src_v[i, sl] = src_v[i, sl] + off
      return c2

    lax.fori_loop(0, CPT, shift_body, 0)

    def zero_acc():
      for t, (t0, tn) in enumerate(_ROWSPLIT):
        @pl.when(sid == t)
        def _(t0=t0, tn=tn):
          pltpu.sync_copy(z_hbm.at[pl.ds(t0, tn)], acc.at[pl.ds(t0, tn)])

    def mult(rows, kk):
      def grp_body(g, c2):
        wg = w_v[kk, pl.ds(g * 16, 16)]
        row0 = g * 16
        for r in range(16):
          wbc = jnp.broadcast_to(wg[r], (16,))
          for j in range(F // 16):
            sl = pl.ds(j * 16, 16)
            rows[row0 + r, sl] = rows[row0 + r, sl] * wbc
        return c2

      lax.fori_loop(0, CH // 16, grp_body, 0)

    def hop(g_hbm):
      pltpu.async_copy(g_hbm.at[src_v.at[0]], rows0, sem0)

      def pair_body(m, carry):
        k0 = 2 * m
        pltpu.async_copy(g_hbm.at[src_v.at[k0 + 1]], rows1, sem1)
        pltpu.make_async_copy(g_hbm.at[src_v.at[k0]], rows0, sem0).wait()
        mult(rows0, k0)
        pltpu.sync_copy(rows0, acc.at[dst_v.at[k0]], add=True)
        knext = lax.rem(k0 + 2, CPT)
        pltpu.async_copy(g_hbm.at[src_v.at[knext]], rows0, sem0)
        pltpu.make_async_copy(g_hbm.at[src_v.at[k0 + 1]], rows1, sem1).wait()
        mult(rows1, k0 + 1)
        pltpu.sync_copy(rows1, acc.at[dst_v.at[k0 + 1]], add=True)
        return carry

      lax.fori_loop(0, CPT // 2, pair_body, 0)
      pltpu.make_async_copy(g_hbm.at[src_v.at[0]], rows0, sem0).wait()
      plsc.subcore_barrier()

    def dump(dst_hbm3):
      for t, (t0, tn) in enumerate(_ROWSPLIT):
        @pl.when(sid == t)
        def _(t0=t0, tn=tn):
          pltpu.sync_copy(acc.at[pl.ds(t0, tn)],
                          dst_hbm3.at[pl.ds(cid * N + t0, tn)])
      plsc.subcore_barrier()

    zero_acc()
    plsc.subcore_barrier()
    hop(x_hbm)
    dump(t1_hbm)
    zero_acc()
    plsc.subcore_barrier()

    hop(t1_hbm)
    for t, (t0, tn) in enumerate(_ROWSPLIT):
      @pl.when(sid == t)
      def _(t0=t0, tn=tn):
        for (o, cn) in _chunks(tn):
          pltpu.sync_copy(acc.at[pl.ds(t0 + o, cn)], rows0.at[pl.ds(0, cn)])
          pltpu.sync_copy(x_hbm.at[pl.ds(cid * N + t0 + o, cn)],
                          rows1.at[pl.ds(0, cn)])

          def cmb_body(r, c2):
            for j in range(F // 16):
              sl = pl.ds(j * 16, 16)
              rows0[r, sl] = 2.0 * rows0[r, sl] - rows1[r, sl]
            return c2

          lax.fori_loop(0, cn, cmb_body, 0)
          pltpu.sync_copy(rows0.at[pl.ds(0, cn)],
                          t2_hbm.at[pl.ds(cid * N + t0 + o, cn)])
    plsc.subcore_barrier()
    zero_acc()
    plsc.subcore_barrier()

    hop(t2_hbm)
    dump(a3_hbm)

  st = jax.ShapeDtypeStruct((N2, F), jnp.float32)
  return pl.kernel(
      body,
      out_type=[st, st, st],
      mesh=mesh,
      compiler_params=pltpu.CompilerParams(use_tc_tiling_on_sc=False),
      scratch_types=[
          pltpu.VMEM((CPT, CH), jnp.int32),
          pltpu.VMEM((CPT, CH), jnp.int32),
          pltpu.VMEM((CPT, CH), jnp.float32),
          pltpu.VMEM((CH, F), jnp.float32),
          pltpu.VMEM((CH, F), jnp.float32),
          pltpu.VMEM_SHARED((N, F), jnp.float32),
          pltpu.SemaphoreType.DMA,
          pltpu.SemaphoreType.DMA,
      ],
  )


@functools.cache
def _make_conv_out(rows, C, F, shortcut, cs, relu, stats):
  grid = (rows // BLK,)

  def body(*refs):
    t0, t1, t2, a3, w, bv = refs[:6]
    i = 6
    if shortcut == "proj":
      s, ws, bsv = refs[i:i + 3]
      i += 3
    elif shortcut == "id":
      s = refs[i]
      i += 1
    y = refs[i]
    i += 1
    if stats:
      ssum, ssq = refs[i:i + 2]
    gi = pl.program_id(0)
    t3 = 2.0 * a3[...] - t1[...]
    acc = (jnp.dot(t0[...], w[0], preferred_element_type=jnp.float32)
           + jnp.dot(t1[...], w[1], preferred_element_type=jnp.float32)
           + jnp.dot(t2[...], w[2], preferred_element_type=jnp.float32)
           + jnp.dot(t3, w[3], preferred_element_type=jnp.float32))
    acc = acc + bv[...]
    if shortcut == "proj":
      acc = acc + jnp.dot(s[...], ws[...], preferred_element_type=jnp.float32) + bsv[...]
    elif shortcut == "id":
      acc = acc + s[...]
    if relu:
      acc = jnp.maximum(acc, 0.0)
    y[...] = acc
    if stats:
      ps = jnp.sum(acc, axis=0, keepdims=True)
      pq = jnp.sum(acc * acc, axis=0, keepdims=True)

      @pl.when(gi == 0)
      def _():
        ssum[...] = ps
        ssq[...] = pq

      @pl.when(gi != 0)
      def _():
        ssum[...] = ssum[...] + ps
        ssq[...] = ssq[...] + pq

  in_specs = [
      pl.BlockSpec((BLK, C), lambda i: (i, 0)),
      pl.BlockSpec((BLK, C), lambda i: (i, 0)),
      pl.BlockSpec((BLK, C), lambda i: (i, 0)),
      pl.BlockSpec((BLK, C), lambda i: (i, 0)),
      pl.BlockSpec((K, C, F), lambda i: (0, 0, 0)),
      pl.BlockSpec((1, F), lambda i: (0, 0)),
  ]
  if shortcut == "proj":
    in_specs += [
        pl.BlockSpec((BLK, cs), lambda i: (i, 0)),
        pl.BlockSpec((cs, F), lambda i: (0, 0)),
        pl.BlockSpec((1, F), lambda i: (0, 0)),
    ]
  elif shortcut == "id":
    in_specs += [pl.BlockSpec((BLK, F), lambda i: (i, 0))]
  out_specs = [pl.BlockSpec((BLK, F), lambda i: (i, 0))]
  out_shape = [jax.ShapeDtypeStruct((rows, F), jnp.float32)]
  if stats:
    out_specs += [pl.BlockSpec((1, F), lambda i: (0, 0))] * 2
    out_shape += [jax.ShapeDtypeStruct((1, F), jnp.float32)] * 2

  return pl.pallas_call(
      body,
      grid=grid,
      in_specs=in_specs,
      out_specs=out_specs,
      out_shape=out_shape,
  )


@functools.cache
def _make_bn_relu(C):
  def body(x_ref, s_ref, q_ref, g_ref, b_ref, o_ref):
    m = s_ref[...] / float(R)
    v = q_ref[...] / float(R) - m * m
    inv = lax.rsqrt(v + EPS)
    o_ref[...] = jnp.maximum((x_ref[...] - m) * inv * g_ref[...] + b_ref[...], 0.0)

  return pl.pallas_call(
      body,
      grid=(R // BLK,),
      in_specs=[
          pl.BlockSpec((BLK, C), lambda i: (i, 0)),
          pl.BlockSpec((1, C), lambda i: (0, 0)),
          pl.BlockSpec((1, C), lambda i: (0, 0)),
          pl.BlockSpec((1, C), lambda i: (0, 0)),
          pl.BlockSpec((1, C), lambda i: (0, 0)),
      ],
      out_specs=pl.BlockSpec((BLK, C), lambda i: (i, 0)),
      out_shape=jax.ShapeDtypeStruct((R, C), jnp.float32),
  )


def _make_head():
  F = 64
  NC = 10
  BLKN = 1000
  grid_n = N // BLKN

  def body(y0, y1, y2, y3, fw, fb, o_ref, mx):
    gi = pl.program_id(0)
    cur = jnp.concatenate(
        [jnp.max(y[...], axis=0, keepdims=True) for y in (y0, y1, y2, y3)], axis=0)

    @pl.when(gi == 0)
    def _():
      mx[...] = cur

    @pl.when(gi != 0)
    def _():
      mx[...] = jnp.maximum(mx[...], cur)

    @pl.when(gi == grid_n - 1)
    def _():
      z = jnp.dot(mx[...], fw[...], preferred_element_type=jnp.float32) + fb[...]
      zm = jnp.max(z, axis=1, keepdims=True)
      e = jnp.exp(z - zm)
      o_ref[...] = (z - zm) - jnp.log(jnp.sum(e, axis=1, keepdims=True))

  return pl.pallas_call(
      body,
      grid=(grid_n,),
      in_specs=[pl.BlockSpec((BLKN, F), lambda i: (i, 0))] * 4 + [
          pl.BlockSpec((F, NC), lambda i: (0, 0)),
          pl.BlockSpec((1, NC), lambda i: (0, 0)),
      ],
      out_specs=pl.BlockSpec((B, NC), lambda i: (0, 0)),
      out_shape=jax.ShapeDtypeStruct((B, NC), jnp.float32),
      scratch_shapes=[pltpu.VMEM((B, F), jnp.float32)],
  )


def kernel(x, params, edge_src, edge_dst, edge_w):
  p = params
  src = edge_src.astype(jnp.int32)
  dst = edge_dst.astype(jnp.int32)
  w = edge_w.astype(jnp.float32)
  padn = EP - E
  pidx = jnp.arange(padn, dtype=jnp.int32) % N
  srcC = jnp.concatenate([src, pidx]).reshape(NCHUNKS, CH)
  dstC = jnp.concatenate([dst, pidx]).reshape(NCHUNKS, CH)
  wC = jnp.concatenate([w, jnp.zeros((padn,), jnp.float32)]).reshape(NCHUNKS, CH)
  zeros = {f: jnp.zeros((N, f), jnp.float32) for f in (32, 64)}

  def cheb3(xt):
    F = xt.shape[1]
    return _make_cheb3(F)(xt, srcC, dstC, wC, zeros[F])

  def conv(xt, C, W, bias, shortcut="none", S=None, Ws=None, bs=None,
           relu=False, stats=True, rows=R):
    T1, T2, A3 = cheb3(xt)
    F = W.shape[2]
    args = [xt.reshape(rows, C), T1.reshape(rows, C), T2.reshape(rows, C),
            A3.reshape(rows, C), W, bias.reshape(1, F)]
    if shortcut == "proj":
      args += [S, Ws, bs.reshape(1, F)]
    elif shortcut == "id":
      args += [S]
    res = _make_conv_out(rows, C, F, shortcut, 0 if S is None else S.shape[1],
                         relu, stats)(*args)
    return res if stats else res[0]

  def bn_relu(h, ss, sq, g, b):
    C = h.shape[1]
    return _make_bn_relu(C)(h, ss, sq, g.reshape(1, C), b.reshape(1, C))

  xt = jnp.pad(jnp.transpose(x, (0, 2, 1)), ((0, 0), (0, 0), (0, 13)))
  xcm = jnp.transpose(xt.reshape(2, 2, N, 16), (0, 2, 1, 3)).reshape(N2, 32)
  W0p = jnp.pad(p['conv0_W'], ((0, 0), (0, 13), (0, 0)))

  out0, s0, q0 = conv(xcm, 16, W0p, p['conv0_b'], relu=True)

  a = bn_relu(out0, s0, q0, p['b1_bn1_g'], p['b1_bn1_b'])
  h1, hs, hq = conv(a.reshape(N2, 32), 16, p['b1_W1'], p['b1_b1'])
  a2 = bn_relu(h1, hs, hq, p['b1_bn2_g'], p['b1_bn2_b'])
  x1, s1, q1 = conv(a2.reshape(N2, 32), 16, p['b1_W2'], p['b1_b2'],
                    shortcut="id", S=out0)

  a = bn_relu(x1, s1, q1, p['b2_bn1_g'], p['b2_bn1_b'])
  h1, hs, hq = conv(a.reshape(N2, 32), 16, p['b2_W1'], p['b2_b1'])
  a2 = bn_relu(h1, hs, hq, p['b2_bn2_g'], p['b2_bn2_b'])
  x2, s2, q2 = conv(a2.reshape(N2, 64), 32, p['b2_W2'], p['b2_b2'],
                    shortcut="proj", S=a, Ws=p['b2_Ws'], bs=p['b2_bs'])

  a = bn_relu(x2, s2, q2, p['b3_bn1_g'], p['b3_bn1_b'])
  h1, hs, hq = conv(a.reshape(N2, 64), 32, p['b3_W1'], p['b3_b1'])
  a2 = bn_relu(h1, hs, hq, p['b3_bn2_g'], p['b3_bn2_b'])

  a2cm = a2.reshape(2, N, 2, 64)
  acm = a.reshape(2, N, 2, 32)
  yj = []
  for j in range(2):
    xt_j = a2cm[:, :, j, :].reshape(N2, 64)
    S_j = acm[:, :, j, :].reshape(N2, 32)
    yj.append(conv(xt_j, 64, p['b3_W2'], p['b3_b2'], shortcut="proj",
                   S=S_j, Ws=p['b3_Ws'], bs=p['b3_bs'], stats=False,
                   rows=N2))

  return _make_head()(yj[0][:N], yj[1][:N], yj[0][N:], yj[1][N:],
                      p['fc_W'], p['fc_b'].reshape(1, 10))


# speedup vs baseline: 1.0862x; 1.0862x over previous
import functools

import jax
import jax.numpy as jnp
from jax import lax
from jax.experimental import pallas as pl
from jax.experimental.pallas import tpu as pltpu
from jax.experimental.pallas import tpu_sc as plsc

N = 10000
E = 160000
B = 4
K = 4
NB = N * B
N2 = 2 * N

CH = 128
EP = 163840
NCHUNKS = EP // CH
NWORKERS = 32
CPT = NCHUNKS // NWORKERS
NTILES = 16
_ROWSPLIT = tuple((t * 632, 632 if t < 15 else N - 15 * 632) for t in range(NTILES))

BLK = 800
BLKN = 1000
EPS = 1e-5


def _chunks(tn):
  out, o = [], 0
  while o < tn:
    c = min(128, tn - o)
    out.append((o, c))
    o += c
  return out


@functools.cache
def _make_spmm(F):
  mesh = plsc.VectorSubcoreMesh(core_axis_name="c", subcore_axis_name="s")

  def body(p0_hbm, p1_hbm, prev_hbm, coef_hbm, src_hbm, dst_hbm, w_hbm, z_hbm,
           tco_hbm, out_hbm, src_v, dst_v, w_v, coef_v, rows0, rows1,
           acc, sem0, sem1):
    cid = lax.axis_index("c")
    sid = lax.axis_index("s")
    wid = sid * 2 + cid
    t0 = sid * 632

    @pl.when(sid < 15)
    def _():
      pltpu.sync_copy(z_hbm.at[pl.ds(t0, 632)], acc.at[pl.ds(t0, 632)])

    @pl.when(sid == 15)
    def _():
      pltpu.sync_copy(z_hbm.at[pl.ds(t0, 520)], acc.at[pl.ds(t0, 520)])
    c0 = wid * CPT
    pltpu.sync_copy(src_hbm.at[pl.ds(c0, CPT)], src_v)
    pltpu.sync_copy(dst_hbm.at[pl.ds(c0, CPT)], dst_v)
    pltpu.sync_copy(w_hbm.at[pl.ds(c0, CPT)], w_v)
    pltpu.sync_copy(coef_hbm, coef_v)
    off = jnp.broadcast_to(cid * N, (16,))

    def shift_body(i, c2):
      for j in range(CH // 16):
        sl = pl.ds(j * 16, 16)
        src_v[i, sl] = src_v[i, sl] + off
      return c2

    lax.fori_loop(0, CPT, shift_body, 0)

    cvec = coef_v[pl.ds(0, 16)]
    abc = jnp.broadcast_to(cvec[0], (16,))
    use_prev = cvec[1]

    def cmb_chunk(o, cn):
      pltpu.sync_copy(p0_hbm.at[pl.ds(o, cn)], rows0.at[pl.ds(0, cn)])
      pltpu.sync_copy(p1_hbm.at[pl.ds(o, cn)], rows1.at[pl.ds(0, cn)])

      def cmb_body(r, c2):
        for j in range(F // 16):
          sl = pl.ds(j * 16, 16)
          rows0[r, sl] = (rows0[r, sl] + rows1[r, sl]) * abc
        return c2

      lax.fori_loop(0, cn, cmb_body, 0)

      @pl.when(use_prev > 0.5)
      def _():
        pltpu.sync_copy(prev_hbm.at[pl.ds(o, cn)], rows1.at[pl.ds(0, cn)])

        def sub_body(r, c2):
          for j in range(F // 16):
            sl = pl.ds(j * 16, 16)
            rows0[r, sl] = rows0[r, sl] - rows1[r, sl]
          return c2

        lax.fori_loop(0, cn, sub_body, 0)

      pltpu.sync_copy(rows0.at[pl.ds(0, cn)],
                      tco_hbm.at[pl.ds(cid * N + o, cn)])

    def cmb4_body(i, c2):
      cmb_chunk(t0 + i * 128, 128)
      return c2

    lax.fori_loop(0, 4, cmb4_body, 0)

    @pl.when(sid < 15)
    def _():
      cmb_chunk(t0 + 512, 120)

    @pl.when(sid == 15)
    def _():
      cmb_chunk(t0 + 512, 8)

    plsc.subcore_barrier()

    def mult(rows, kk):
      def grp_body(g, c2):
        wg = w_v[kk, pl.ds(g * 16, 16)]
        row0 = g * 16
        for r in range(16):
          wbc = jnp.broadcast_to(wg[r], (16,))
          for j in range(F // 16):
            sl = pl.ds(j * 16, 16)
            rows[row0 + r, sl] = rows[row0 + r, sl] * wbc
        return c2

      lax.fori_loop(0, CH // 16, grp_body, 0)

    pltpu.async_copy(tco_hbm.at[src_v.at[0]], rows0, sem0)

    def pair_body(m, carry):
      k0 = 2 * m
      pltpu.async_copy(tco_hbm.at[src_v.at[k0 + 1]], rows1, sem1)
      pltpu.make_async_copy(tco_hbm.at[src_v.at[k0]], rows0, sem0).wait()
      mult(rows0, k0)
      pltpu.sync_copy(rows0, acc.at[dst_v.at[k0]], add=True)
      knext = lax.rem(k0 + 2, CPT)
      pltpu.async_copy(tco_hbm.at[src_v.at[knext]], rows0, sem0)
      pltpu.make_async_copy(tco_hbm.at[src_v.at[k0 + 1]], rows1, sem1).wait()
      mult(rows1, k0 + 1)
      pltpu.sync_copy(rows1, acc.at[dst_v.at[k0 + 1]], add=True)
      return carry

    lax.fori_loop(0, CPT // 2, pair_body, 0)
    pltpu.make_async_copy(tco_hbm.at[src_v.at[0]], rows0, sem0).wait()
    plsc.subcore_barrier()

    @pl.when(sid < 15)
    def _():
      pltpu.sync_copy(acc.at[pl.ds(t0, 632)], out_hbm.at[cid, pl.ds(t0, 632)])

    @pl.when(sid == 15)
    def _():
      pltpu.sync_copy(acc.at[pl.ds(t0, 520)], out_hbm.at[cid, pl.ds(t0, 520)])

  return pl.kernel(
      body,
      out_type=[jax.ShapeDtypeStruct((N2, F), jnp.float32),
                jax.ShapeDtypeStruct((2, N, F), jnp.float32)],
      mesh=mesh,
      compiler_params=pltpu.CompilerParams(use_tc_tiling_on_sc=False),
      scratch_types=[
          pltpu.VMEM((CPT, CH), jnp.int32),
          pltpu.VMEM((CPT, CH), jnp.int32),
          pltpu.VMEM((CPT, CH), jnp.float32),
          pltpu.VMEM((16,), jnp.float32),
          pltpu.VMEM((CH, F), jnp.float32),
          pltpu.VMEM((CH, F), jnp.float32),
          pltpu.VMEM_SHARED((N, F), jnp.float32),
          pltpu.SemaphoreType.DMA,
          pltpu.SemaphoreType.DMA,
      ],
  )


@functools.cache
def _make_conv_out(rows, C, F, shortcut, cs, relu, stats):
  grid = (rows // BLK,)

  def body(*refs):
    t0, t1, t2, p3, w, bv = refs[:6]
    i = 6
    if shortcut == "proj":
      s, ws, bsv = refs[i:i + 3]
      i += 3
    elif shortcut == "id":
      s = refs[i]
      i += 1
    y = refs[i]
    i += 1
    if stats:
      ssum, ssq = refs[i:i + 2]
    gi = pl.program_id(0)
    t3 = 2.0 * (p3[0] + p3[1]) - t1[...]
    acc = (jnp.dot(t0[...], w[0], preferred_element_type=jnp.float32)
           + jnp.dot(t1[...], w[1], preferred_element_type=jnp.float32)
           + jnp.dot(t2[...], w[2], preferred_element_type=jnp.float32)
           + jnp.dot(t3, w[3], preferred_element_type=jnp.float32))
    acc = acc + bv[...]
    if shortcut == "proj":
      acc = acc + jnp.dot(s[...], ws[...], preferred_element_type=jnp.float32) + bsv[...]
    elif shortcut == "id":
      acc = acc + s[...]
    if relu:
      acc = jnp.maximum(acc, 0.0)
    y[...] = acc
    if stats:
      ps = jnp.sum(acc, axis=0, keepdims=True)
      pq = jnp.sum(acc * acc, axis=0, keepdims=True)

      @pl.when(gi == 0)
      def _():
        ssum[...] = ps
        ssq[...] = pq

      @pl.when(gi != 0)
      def _():
        ssum[...] = ssum[...] + ps
        ssq[...] = ssq[...] + pq

  in_specs = [
      pl.BlockSpec((BLK, C), lambda i: (i, 0)),
      pl.BlockSpec((BLK, C), lambda i: (i, 0)),
      pl.BlockSpec((BLK, C), lambda i: (i, 0)),
      pl.BlockSpec((2, BLK, C), lambda i: (0, i, 0)),
      pl.BlockSpec((K, C, F), lambda i: (0, 0, 0)),
      pl.BlockSpec((1, F), lambda i: (0, 0)),
  ]
  if shortcut == "proj":
    in_specs += [
        pl.BlockSpec((BLK, cs), lambda i: (i, 0)),
        pl.BlockSpec((cs, F), lambda i: (0, 0)),
        pl.BlockSpec((1, F), lambda i: (0, 0)),
    ]
  elif shortcut == "id":
    in_specs += [pl.BlockSpec((BLK, F), lambda i: (i, 0))]
  out_specs = [pl.BlockSpec((BLK, F), lambda i: (i, 0))]
  out_shape = [jax.ShapeDtypeStruct((rows, F), jnp.float32)]
  if stats:
    out_specs += [pl.BlockSpec((1, F), lambda i: (0, 0))] * 2
    out_shape += [jax.ShapeDtypeStruct((1, F), jnp.float32)] * 2

  return pl.pallas_call(
      body,
      grid=grid,
      in_specs=in_specs,
      out_specs=out_specs,
      out_shape=out_shape,
  )


@functools.cache
def _make_bn_relu(C):
  def body(x_ref, s_ref, q_ref, g_ref, b_ref, o_ref):
    m = s_ref[...] / float(NB)
    v = q_ref[...] / float(NB) - m * m
    inv = lax.rsqrt(v + EPS)
    o_ref[...] = jnp.maximum((x_ref[...] - m) * inv * g_ref[...] + b_ref[...], 0.0)

  return pl.pallas_call(
      body,
      grid=(NB // BLK,),
      in_specs=[
          pl.BlockSpec((BLK, C), lambda i: (i, 0)),
          pl.BlockSpec((1, C), lambda i: (0, 0)),
          pl.BlockSpec((1, C), lambda i: (0, 0)),
          pl.BlockSpec((1, C), lambda i: (0, 0)),
          pl.BlockSpec((1, C), lambda i: (0, 0)),
      ],
      out_specs=pl.BlockSpec((BLK, C), lambda i: (i, 0)),
      out_shape=jax.ShapeDtypeStruct((NB, C), jnp.float32),
  )


def _make_head():
  F = 64
  NC = 10
  grid_n = N // BLKN

  def body(y0, y1, y2, y3, fw, fb, o_ref, mx):
    gi = pl.program_id(0)
    cur = jnp.concatenate(
        [jnp.max(y[...], axis=0, keepdims=True) for y in (y0, y1, y2, y3)], axis=0)

    @pl.when(gi == 0)
    def _():
      mx[...] = cur

    @pl.when(gi != 0)
    def _():
      mx[...] = jnp.maximum(mx[...], cur)

    @pl.when(gi == grid_n - 1)
    def _():
      z = jnp.dot(mx[...], fw[...], preferred_element_type=jnp.float32) + fb[...]
      zm = jnp.max(z, axis=1, keepdims=True)
      e = jnp.exp(z - zm)
      o_ref[...] = (z - zm) - jnp.log(jnp.sum(e, axis=1, keepdims=True))

  return pl.pallas_call(
      body,
      grid=(grid_n,),
      in_specs=[pl.BlockSpec((BLKN, F), lambda i: (i, 0))] * 4 + [
          pl.BlockSpec((F, NC), lambda i: (0, 0)),
          pl.BlockSpec((1, NC), lambda i: (0, 0)),
      ],
      out_specs=pl.BlockSpec((B, NC), lambda i: (0, 0)),
      out_shape=jax.ShapeDtypeStruct((B, NC), jnp.float32),
      scratch_shapes=[pltpu.VMEM((B, F), jnp.float32)],
  )


def kernel(x, params, edge_src, edge_dst, edge_w):
  p = params
  src = edge_src.astype(jnp.int32)
  dst = edge_dst.astype(jnp.int32)
  w = edge_w.astype(jnp.float32)
  padn = EP - E
  pidx = jnp.arange(padn, dtype=jnp.int32) % N
  srcC = jnp.concatenate([src, pidx]).reshape(NCHUNKS, CH)
  dstC = jnp.concatenate([dst, pidx]).reshape(NCHUNKS, CH)
  wC = jnp.concatenate([w, jnp.zeros((padn,), jnp.float32)]).reshape(NCHUNKS, CH)
  zeros = {f: jnp.zeros((N, f), jnp.float32) for f in (16, 64, 128)}
  cf_first = jnp.array([1.0, 0.0] + [0.0] * 14, jnp.float32)
  cf_mid = jnp.array([1.0, 0.0] + [0.0] * 14, jnp.float32)
  cf_last = jnp.array([2.0, 1.0] + [0.0] * 14, jnp.float32)

  def spmm(p0, p1, prev, cf):
    F = p0.shape[1]
    return _make_spmm(F)(p0, p1, prev, cf, srcC, dstC, wC, zeros[F])

  def cheb_T(xt):
    F = xt.shape[1]
    z = zeros[F]
    _, P1 = spmm(xt, z, z, cf_first)
    T1c, P2 = spmm(P1[0], P1[1], z, cf_mid)
    T2c, P3 = spmm(P2[0], P2[1], xt, cf_last)
    return xt, T1c[:N], T2c[:N], P3

  def conv(xt, rows, C, W, bias, shortcut="none", S=None, Ws=None, bs=None,
           relu=False, stats=True):
    T0, T1, T2, P3 = cheb_T(xt)
    F = W.shape[2]
    args = [T0.reshape(rows, C), T1.reshape(rows, C), T2.reshape(rows, C),
            P3.reshape(2, rows, C), W, bias.reshape(1, F)]
    if shortcut == "proj":
      args += [S, Ws, bs.reshape(1, F)]
    elif shortcut == "id":
      args += [S]
    res = _make_conv_out(rows, C, F, shortcut, 0 if S is None else S.shape[1],
                         relu, stats)(*args)
    return res if stats else res[0]

  def bn_relu(h, ss, sq, g, b):
    C = h.shape[1]
    return _make_bn_relu(C)(h, ss, sq, g.reshape(1, C), b.reshape(1, C))

  xt16 = jnp.pad(jnp.transpose(x, (2, 0, 1)), ((0, 0), (0, 0), (0, 1))).reshape(N, 16)
  W0p = jnp.pad(p['conv0_W'], ((0, 0), (0, 1), (0, 0)))

  out0, s0, q0 = conv(xt16, NB, 4, W0p, p['conv0_b'], relu=True)

  a = bn_relu(out0, s0, q0, p['b1_bn1_g'], p['b1_bn1_b'])
  h1, hs, hq = conv(a.reshape(N, 64), NB, 16, p['b1_W1'], p['b1_b1'])
  a2 = bn_relu(h1, hs, hq, p['b1_bn2_g'], p['b1_bn2_b'])
  x1, s1, q1 = conv(a2.reshape(N, 64), NB, 16, p['b1_W2'], p['b1_b2'],
                    shortcut="id", S=out0)

  a = bn_relu(x1, s1, q1, p['b2_bn1_g'], p['b2_bn1_b'])
  h1, hs, hq = conv(a.reshape(N, 64), NB, 16, p['b2_W1'], p['b2_b1'])
  a2 = bn_relu(h1, hs, hq, p['b2_bn2_g'], p['b2_bn2_b'])
  x2, s2, q2 = conv(a2.reshape(N, 128), NB, 32, p['b2_W2'], p['b2_b2'],
                    shortcut="proj", S=a, Ws=p['b2_Ws'], bs=p['b2_bs'])

  a = bn_relu(x2, s2, q2, p['b3_bn1_g'], p['b3_bn1_b'])
  h1, hs, hq = conv(a.reshape(N, 128), NB, 32, p['b3_W1'], p['b3_b1'])
  a2 = bn_relu(h1, hs, hq, p['b3_bn2_g'], p['b3_bn2_b'])

  a2r = a2.reshape(N, B, 64)
  ar = a.reshape(N, B, 32)
  ybs = []
  for h in range(2):
    xt_h = a2r[:, 2 * h:2 * h + 2, :].reshape(N, 128)
    S_h = ar[:, 2 * h:2 * h + 2, :].reshape(2 * N, 32)
    y_h = conv(xt_h, 2 * N, 64, p['b3_W2'], p['b3_b2'],
               shortcut="proj", S=S_h, Ws=p['b3_Ws'], bs=p['b3_bs'], stats=False)
    yr = y_h.reshape(N, 2, 64)
    ybs += [yr[:, 0, :], yr[:, 1, :]]

  return _make_head()(ybs[0], ybs[1], ybs[2], ybs[3],
                      p['fc_W'], p['fc_b'].reshape(1, 10))


# speedup vs baseline: 1.2922x; 1.1896x over previous
import functools

import jax
import jax.numpy as jnp
from jax import lax
from jax.experimental import pallas as pl
from jax.experimental.pallas import tpu as pltpu
from jax.experimental.pallas import tpu_sc as plsc

N = 10000
E = 160000
B = 4
K = 4
NB = N * B

CH = 128
EP = 163840
NCHUNKS = EP // CH
NWORKERS = 32
CPT = NCHUNKS // NWORKERS
NTILES = 16
_ROWSPLIT = tuple((t * 632, 632 if t < 15 else N - 15 * 632) for t in range(NTILES))

BLK = 800
BLKN = 1000
EPS = 1e-5


@functools.cache
def _make_spmm(F):
  mesh = plsc.VectorSubcoreMesh(core_axis_name="c", subcore_axis_name="s")

  def body(x_hbm, src_hbm, dst_hbm, w_hbm, z_hbm, out_hbm,
           src_v, dst_v, w_v, rows0, rows1, acc, sem0, sem1):
    cid = lax.axis_index("c")
    sid = lax.axis_index("s")
    wid = sid * 2 + cid
    for t, (t0, tn) in enumerate(_ROWSPLIT):
      @pl.when(sid == t)
      def _(t0=t0, tn=tn):
        pltpu.sync_copy(z_hbm.at[pl.ds(t0, tn)], acc.at[pl.ds(t0, tn)])
    c0 = wid * CPT
    pltpu.sync_copy(src_hbm.at[pl.ds(c0, CPT)], src_v)
    pltpu.sync_copy(dst_hbm.at[pl.ds(c0, CPT)], dst_v)
    pltpu.sync_copy(w_hbm.at[pl.ds(c0, CPT)], w_v)
    plsc.subcore_barrier()

    def mult(rows, kk):
      def grp_body(g, c2):
        wg = w_v[kk, pl.ds(g * 16, 16)]
        row0 = g * 16
        for r in range(16):
          wbc = jnp.broadcast_to(wg[r], (16,))
          for j in range(F // 16):
            sl = pl.ds(j * 16, 16)
            rows[row0 + r, sl] = rows[row0 + r, sl] * wbc
        return c2

      lax.fori_loop(0, CH // 16, grp_body, 0)

    pltpu.async_copy(x_hbm.at[src_v.at[0]], rows0, sem0)

    def pair_body(m, carry):
      k0 = 2 * m
      pltpu.async_copy(x_hbm.at[src_v.at[k0 + 1]], rows1, sem1)
      pltpu.make_async_copy(x_hbm.at[src_v.at[k0]], rows0, sem0).wait()
      mult(rows0, k0)
      pltpu.sync_copy(rows0, acc.at[dst_v.at[k0]], add=True)
      knext = lax.rem(k0 + 2, CPT)
      pltpu.async_copy(x_hbm.at[src_v.at[knext]], rows0, sem0)
      pltpu.make_async_copy(x_hbm.at[src_v.at[k0 + 1]], rows1, sem1).wait()
      mult(rows1, k0 + 1)
      pltpu.sync_copy(rows1, acc.at[dst_v.at[k0 + 1]], add=True)
      return carry

    lax.fori_loop(0, CPT // 2, pair_body, 0)
    pltpu.make_async_copy(x_hbm.at[src_v.at[0]], rows0, sem0).wait()
    plsc.subcore_barrier()
    for t, (t0, tn) in enumerate(_ROWSPLIT):
      @pl.when(sid == t)
      def _(t0=t0, tn=tn):
        pltpu.sync_copy(acc.at[pl.ds(t0, tn)], out_hbm.at[cid, pl.ds(t0, tn)])

  return pl.kernel(
      body,
      out_type=jax.ShapeDtypeStruct((2, N, F), jnp.float32),
      mesh=mesh,
      compiler_params=pltpu.CompilerParams(use_tc_tiling_on_sc=False),
      scratch_types=[
          pltpu.VMEM((CPT, CH), jnp.int32),
          pltpu.VMEM((CPT, CH), jnp.int32),
          pltpu.VMEM((CPT, CH), jnp.float32),
          pltpu.VMEM((CH, F), jnp.float32),
          pltpu.VMEM((CH, F), jnp.float32),
          pltpu.VMEM_SHARED((N, F), jnp.float32),
          pltpu.SemaphoreType.DMA,
          pltpu.SemaphoreType.DMA,
      ],
  )


@functools.cache
def _make_cheb_first(F):
  def body(p_ref, o_ref):
    o_ref[...] = p_ref[0] + p_ref[1]

  return pl.pallas_call(
      body,
      grid=(N // BLKN,),
      in_specs=[pl.BlockSpec((2, BLKN, F), lambda i: (0, i, 0))],
      out_specs=pl.BlockSpec((BLKN, F), lambda i: (i, 0)),
      out_shape=jax.ShapeDtypeStruct((N, F), jnp.float32),
  )


@functools.cache
def _make_cheb_next(F):
  def body(p_ref, t_ref, o_ref):
    o_ref[...] = 2.0 * (p_ref[0] + p_ref[1]) - t_ref[...]

  return pl.pallas_call(
      body,
      grid=(N // BLKN,),
      in_specs=[
          pl.BlockSpec((2, BLKN, F), lambda i: (0, i, 0)),
          pl.BlockSpec((BLKN, F), lambda i: (i, 0)),
      ],
      out_specs=pl.BlockSpec((BLKN, F), lambda i: (i, 0)),
      out_shape=jax.ShapeDtypeStruct((N, F), jnp.float32),
  )


@functools.cache
def _make_conv_out(rows, C, F, shortcut, cs, relu, stats):
  grid = (rows // BLK,)

  def body(*refs):
    t0, t1, t2, p3, w, bv = refs[:6]
    i = 6
    if shortcut == "proj":
      s, ws, bsv = refs[i:i + 3]
      i += 3
    elif shortcut == "id":
      s = refs[i]
      i += 1
    y = refs[i]
    i += 1
    if stats:
      ssum, ssq = refs[i:i + 2]
    gi = pl.program_id(0)
    t3 = 2.0 * (p3[0] + p3[1]) - t1[...]
    acc = (jnp.dot(t0[...], w[0], preferred_element_type=jnp.float32)
           + jnp.dot(t1[...], w[1], preferred_element_type=jnp.float32)
           + jnp.dot(t2[...], w[2], preferred_element_type=jnp.float32)
           + jnp.dot(t3, w[3], preferred_element_type=jnp.float32))
    acc = acc + bv[...]
    if shortcut == "proj":
      acc = acc + jnp.dot(s[...], ws[...], preferred_element_type=jnp.float32) + bsv[...]
    elif shortcut == "id":
      acc = acc + s[...]
    if relu:
      acc = jnp.maximum(acc, 0.0)
    y[...] = acc
    if stats:
      ps = jnp.sum(acc, axis=0, keepdims=True)
      pq = jnp.sum(acc * acc, axis=0, keepdims=True)

      @pl.when(gi == 0)
      def _():
        ssum[...] = ps
        ssq[...] = pq

      @pl.when(gi != 0)
      def _():
        ssum[...] = ssum[...] + ps
        ssq[...] = ssq[...] + pq

  in_specs = [
      pl.BlockSpec((BLK, C), lambda i: (i, 0)),
      pl.BlockSpec((BLK, C), lambda i: (i, 0)),
      pl.BlockSpec((BLK, C), lambda i: (i, 0)),
      pl.BlockSpec((2, BLK, C), lambda i: (0, i, 0)),
      pl.BlockSpec((K, C, F), lambda i: (0, 0, 0)),
      pl.BlockSpec((1, F), lambda i: (0, 0)),
  ]
  if shortcut == "proj":
    in_specs += [
        pl.BlockSpec((BLK, cs), lambda i: (i, 0)),
        pl.BlockSpec((cs, F), lambda i: (0, 0)),
        pl.BlockSpec((1, F), lambda i: (0, 0)),
    ]
  elif shortcut == "id":
    in_specs += [pl.BlockSpec((BLK, F), lambda i: (i, 0))]
  out_specs = [pl.BlockSpec((BLK, F), lambda i: (i, 0))]
  out_shape = [jax.ShapeDtypeStruct((rows, F), jnp.float32)]
  if stats:
    out_specs += [pl.BlockSpec((1, F), lambda i: (0, 0))] * 2
    out_shape += [jax.ShapeDtypeStruct((1, F), jnp.float32)] * 2

  return pl.pallas_call(
      body,
      grid=grid,
      in_specs=in_specs,
      out_specs=out_specs,
      out_shape=out_shape,
  )


@functools.cache
def _make_bn_relu(C):
  def body(x_ref, s_ref, q_ref, g_ref, b_ref, o_ref):
    m = s_ref[...] / float(NB)
    v = q_ref[...] / float(NB) - m * m
    inv = lax.rsqrt(v + EPS)
    o_ref[...] = jnp.maximum((x_ref[...] - m) * inv * g_ref[...] + b_ref[...], 0.0)

  return pl.pallas_call(
      body,
      grid=(NB // BLK,),
      in_specs=[
          pl.BlockSpec((BLK, C), lambda i: (i, 0)),
          pl.BlockSpec((1, C), lambda i: (0, 0)),
          pl.BlockSpec((1, C), lambda i: (0, 0)),
          pl.BlockSpec((1, C), lambda i: (0, 0)),
          pl.BlockSpec((1, C), lambda i: (0, 0)),
      ],
      out_specs=pl.BlockSpec((BLK, C), lambda i: (i, 0)),
      out_shape=jax.ShapeDtypeStruct((NB, C), jnp.float32),
  )


def _make_head():
  F = 64
  NC = 10
  grid_n = N // BLKN

  def body(y0, y1, y2, y3, fw, fb, o_ref, mx):
    gi = pl.program_id(0)
    cur = jnp.concatenate(
        [jnp.max(y[...], axis=0, keepdims=True) for y in (y0, y1, y2, y3)], axis=0)

    @pl.when(gi == 0)
    def _():
      mx[...] = cur

    @pl.when(gi != 0)
    def _():
      mx[...] = jnp.maximum(mx[...], cur)

    @pl.when(gi == grid_n - 1)
    def _():
      z = jnp.dot(mx[...], fw[...], preferred_element_type=jnp.float32) + fb[...]
      zm = jnp.max(z, axis=1, keepdims=True)
      e = jnp.exp(z - zm)
      o_ref[...] = (z - zm) - jnp.log(jnp.sum(e, axis=1, keepdims=True))

  return pl.pallas_call(
      body,
      grid=(grid_n,),
      in_specs=[pl.BlockSpec((BLKN, F), lambda i: (i, 0))] * 4 + [
          pl.BlockSpec((F, NC), lambda i: (0, 0)),
          pl.BlockSpec((1, NC), lambda i: (0, 0)),
      ],
      out_specs=pl.BlockSpec((B, NC), lambda i: (0, 0)),
      out_shape=jax.ShapeDtypeStruct((B, NC), jnp.float32),
      scratch_shapes=[pltpu.VMEM((B, F), jnp.float32)],
  )


def kernel(x, params, edge_src, edge_dst, edge_w):
  p = params
  src = edge_src.astype(jnp.int32)
  dst = edge_dst.astype(jnp.int32)
  w = edge_w.astype(jnp.float32)
  padn = EP - E
  pidx = jnp.arange(padn, dtype=jnp.int32) % N
  srcC = jnp.concatenate([src, pidx]).reshape(NCHUNKS, CH)
  dstC = jnp.concatenate([dst, pidx]).reshape(NCHUNKS, CH)
  wC = jnp.concatenate([w, jnp.zeros((padn,), jnp.float32)]).reshape(NCHUNKS, CH)
  zeros = {f: jnp.zeros((N, f), jnp.float32) for f in (16, 64, 128)}

  def spmm(xt):
    return _make_spmm(xt.shape[1])(xt, srcC, dstC, wC, zeros[xt.shape[1]])

  def cheb_T(xt):
    P1 = spmm(xt)
    T1 = _make_cheb_first(xt.shape[1])(P1)
    P2 = spmm(T1)
    T2 = _make_cheb_next(xt.shape[1])(P2, xt)
    P3 = spmm(T2)
    return xt, T1, T2, P3

  def conv(xt, rows, C, W, bias, shortcut="none", S=None, Ws=None, bs=None,
           relu=False, stats=True):
    T0, T1, T2, P3 = cheb_T(xt)
    F = W.shape[2]
    args = [T0.reshape(rows, C), T1.reshape(rows, C), T2.reshape(rows, C),
            P3.reshape(2, rows, C), W, bias.reshape(1, F)]
    if shortcut == "proj":
      args += [S, Ws, bs.reshape(1, F)]
    elif shortcut == "id":
      args += [S]
    res = _make_conv_out(rows, C, F, shortcut, 0 if S is None else S.shape[1],
                         relu, stats)(*args)
    return res if stats else res[0]

  def bn_relu(h, ss, sq, g, b):
    C = h.shape[1]
    return _make_bn_relu(C)(h, ss, sq, g.reshape(1, C), b.reshape(1, C))

  xt16 = jnp.pad(jnp.transpose(x, (2, 0, 1)), ((0, 0), (0, 0), (0, 1))).reshape(N, 16)
  W0p = jnp.pad(p['conv0_W'], ((0, 0), (0, 1), (0, 0)))

  out0, s0, q0 = conv(xt16, NB, 4, W0p, p['conv0_b'], relu=True)

  a = bn_relu(out0, s0, q0, p['b1_bn1_g'], p['b1_bn1_b'])
  h1, hs, hq = conv(a.reshape(N, 64), NB, 16, p['b1_W1'], p['b1_b1'])
  a2 = bn_relu(h1, hs, hq, p['b1_bn2_g'], p['b1_bn2_b'])
  x1, s1, q1 = conv(a2.reshape(N, 64), NB, 16, p['b1_W2'], p['b1_b2'],
                    shortcut="id", S=out0)

  a = bn_relu(x1, s1, q1, p['b2_bn1_g'], p['b2_bn1_b'])
  h1, hs, hq = conv(a.reshape(N, 64), NB, 16, p['b2_W1'], p['b2_b1'])
  a2 = bn_relu(h1, hs, hq, p['b2_bn2_g'], p['b2_bn2_b'])
  x2, s2, q2 = conv(a2.reshape(N, 128), NB, 32, p['b2_W2'], p['b2_b2'],
                    shortcut="proj", S=a, Ws=p['b2_Ws'], bs=p['b2_bs'])

  a = bn_relu(x2, s2, q2, p['b3_bn1_g'], p['b3_bn1_b'])
  h1, hs, hq = conv(a.reshape(N, 128), NB, 32, p['b3_W1'], p['b3_b1'])
  a2 = bn_relu(h1, hs, hq, p['b3_bn2_g'], p['b3_bn2_b'])

  a2r = a2.reshape(N, B, 64)
  ar = a.reshape(N, B, 32)
  ybs = []
  for h in range(2):
    xt_h = a2r[:, 2 * h:2 * h + 2, :].reshape(N, 128)
    S_h = ar[:, 2 * h:2 * h + 2, :].reshape(2 * N, 32)
    y_h = conv(xt_h, 2 * N, 64, p['b3_W2'], p['b3_b2'],
               shortcut="proj", S=S_h, Ws=p['b3_Ws'], bs=p['b3_bs'], stats=False)
    yr = y_h.reshape(N, 2, 64)
    ybs += [yr[:, 0, :], yr[:, 1, :]]

  return _make_head()(ybs[0], ybs[1], ybs[2], ybs[3],
                      p['fc_W'], p['fc_b'].reshape(1, 10))
